# baseline (speedup 1.0000x reference)
import functools

import jax
import jax.numpy as jnp
from jax import lax
from jax.experimental import pallas as pl
from jax.experimental.pallas import tpu as pltpu
from jax.experimental.pallas import tpu_sc as plsc

EMBED_DIM = 64
NUM_CLASSES = 1000000
BATCH = 16384
LAMBDA_MCE = 1.0
SCALE = 8.0

_CB = 512
_TGRID = (NUM_CLASSES + 2 * _CB - 1) // (2 * _CB)
_PAIR_ROWS = _TGRID * _CB

_info = plsc.get_sparse_core_info()
_NC, _NS = _info.num_cores, _info.num_subcores
_NW = _NC * _NS
_B_PER_W = BATCH // _NW
_G = 16
_NGROUP = _B_PER_W // _G
_CHUNK = 128
_NCHUNK = _B_PER_W // _CHUNK

_mesh = plsc.VectorSubcoreMesh(core_axis_name="c", subcore_axis_name="s")


def _rsqrt(x):
    i = plsc.bitcast(x, jnp.int32)
    i = 0x5F3759DF - lax.shift_right_arithmetic(i, 1)
    y = plsc.bitcast(i, jnp.float32)
    for _ in range(3):
        y = y * (1.5 - 0.5 * x * y * y)
    return y


@functools.partial(
    pl.kernel,
    mesh=_mesh,
    out_type=jax.ShapeDtypeStruct((2, BATCH), jnp.float32),
    scratch_types=[
        pltpu.VMEM((_B_PER_W,), jnp.int32),
        pltpu.VMEM((_NCHUNK, _CHUNK), jnp.int32),
        pltpu.VMEM((EMBED_DIM, _B_PER_W), jnp.float32),
        pltpu.VMEM((_B_PER_W, 128), jnp.float32),
        pltpu.VMEM((_B_PER_W,), jnp.float32),
        pltpu.VMEM((_B_PER_W,), jnp.float32),
        pltpu.SemaphoreType.DMA,
    ],
    compiler_params=pltpu.CompilerParams(needs_layout_passes=False),
)
def _sc_fused(pairs_hbm, feat_t, labels_hbm, streak_hbm, out_hbm,
              labels_v, idx_v, featv, rows_v, streak_v, outv, sem):
    wid = lax.axis_index("s") * _NC + lax.axis_index("c")
    base = wid * _B_PER_W

    pltpu.sync_copy(labels_hbm.at[pl.ds(base, _B_PER_W)], labels_v)
    lane = lax.iota(jnp.int32, _G)
    for j in range(_NCHUNK):
        for k in range(_CHUNK // _G):
            lbl = labels_v[pl.ds(j * _CHUNK + k * _G, _G)]
            pr = lax.shift_left(lax.shift_right_logical(lbl, 10), 9)
            idx_v[j, pl.ds(k * _G, _G)] = pr | (lbl & 511)
    copies = []
    for j in range(_NCHUNK):
        copies.append(
            pltpu.async_copy(
                pairs_hbm.at[idx_v.at[j]],
                rows_v.at[pl.ds(j * _CHUNK, _CHUNK)],
                sem,
            )
        )
    pltpu.sync_copy(feat_t.at[:, pl.ds(base, _B_PER_W)], featv)
    pltpu.sync_copy(streak_hbm.at[pl.ds(base, _B_PER_W)], streak_v)
    for c in copies:
        c.wait()

    def group(g, _):
        loc = g * _G
        slot = loc + lane
        half = (lax.shift_right_logical(labels_v[pl.ds(loc, _G)], 9) & 1) * EMBED_DIM
        dot = jnp.zeros((_G,), jnp.float32)
        nrm = jnp.zeros((_G,), jnp.float32)
        for d in range(EMBED_DIM):
            e = plsc.load_gather(rows_v, [slot, half + d])
            f = featv[d, pl.ds(loc, _G)]
            dot = dot + e * f
            nrm = nrm + f * f
        ns = jnp.maximum(nrm, 1e-30)
        norm = jnp.maximum(ns * _rsqrt(ns), 1e-6)
        alpha = 1.0 / (1.0 + jnp.exp(-(dot / (SCALE * norm))))
        amp = 1.0 + LAMBDA_MCE * (1.0 - alpha) * streak_v[pl.ds(loc, _G)]
        outv[pl.ds(loc, _G)] = amp
        return ()

    lax.fori_loop(0, _NGROUP, group, (), unroll=False)

    pltpu.sync_copy(outv, out_hbm.at[0, pl.ds(base, _B_PER_W)])
    pltpu.sync_copy(outv, out_hbm.at[1, pl.ds(base, _B_PER_W)])


def _tc_pairs_body(x1_ref, x2_ref, o_ref):
    o_ref[:, 0:EMBED_DIM] = jnp.transpose(x1_ref[...])
    o_ref[:, EMBED_DIM:2 * EMBED_DIM] = jnp.transpose(x2_ref[...])


def _tc_pairs(table_t):
    return pl.pallas_call(
        _tc_pairs_body,
        grid=(_TGRID,),
        in_specs=[
            pl.BlockSpec((EMBED_DIM, _CB), lambda i: (0, 2 * i)),
            pl.BlockSpec((EMBED_DIM, _CB), lambda i: (0, 2 * i + 1)),
        ],
        out_specs=pl.BlockSpec((_CB, 2 * EMBED_DIM), lambda i: (i, 0)),
        out_shape=jax.ShapeDtypeStruct((_PAIR_ROWS, 2 * EMBED_DIM), jnp.float32),
    )(table_t, table_t)


def kernel(gt_labels, pooled_features, streak_ratio, table):
    table_t = jnp.swapaxes(table, 0, 1)
    pairs = _tc_pairs(table_t)
    feat_t = jnp.swapaxes(pooled_features, 0, 1)
    return _sc_fused(pairs, feat_t, gt_labels, streak_ratio)


# speedup vs baseline: 1.7298x; 1.7298x over previous
import functools

import jax
import jax.numpy as jnp
from jax import lax
from jax.experimental import pallas as pl
from jax.experimental.pallas import tpu as pltpu
from jax.experimental.pallas import tpu_sc as plsc

EMBED_DIM = 64
NUM_CLASSES = 1000000
BATCH = 16384
LAMBDA_MCE = 1.0
SCALE = 8.0

_CB = 1024
_TGRID = (NUM_CLASSES + 2 * _CB - 1) // (2 * _CB)
_PAIR_ROWS = _TGRID * _CB

_info = plsc.get_sparse_core_info()
_NC, _NS = _info.num_cores, _info.num_subcores
_NW = _NC * _NS
_B_PER_W = BATCH // _NW
_G = 16
_NGROUP = _B_PER_W // _G
_CHUNK = 128
_NCHUNK = _B_PER_W // _CHUNK

_mesh = plsc.VectorSubcoreMesh(core_axis_name="c", subcore_axis_name="s")


def _rsqrt(x):
    i = plsc.bitcast(x, jnp.int32)
    i = 0x5F3759DF - lax.shift_right_arithmetic(i, 1)
    y = plsc.bitcast(i, jnp.float32)
    for _ in range(3):
        y = y * (1.5 - 0.5 * x * y * y)
    return y


@functools.partial(
    pl.kernel,
    mesh=_mesh,
    out_type=jax.ShapeDtypeStruct((2, BATCH), jnp.float32),
    scratch_types=[
        pltpu.VMEM((_B_PER_W,), jnp.int32),
        pltpu.VMEM((_NCHUNK, _CHUNK), jnp.int32),
        pltpu.VMEM((EMBED_DIM, _B_PER_W), jnp.float32),
        pltpu.VMEM((_B_PER_W, 128), jnp.float32),
        pltpu.VMEM((_B_PER_W,), jnp.float32),
        pltpu.VMEM((_B_PER_W,), jnp.float32),
        pltpu.SemaphoreType.DMA,
    ],
    compiler_params=pltpu.CompilerParams(needs_layout_passes=False),
)
def _sc_fused(pairs_hbm, feat_t, labels_hbm, streak_hbm, out_hbm,
              labels_v, idx_v, featv, rows_v, streak_v, outv, sem):
    wid = lax.axis_index("s") * _NC + lax.axis_index("c")
    base = wid * _B_PER_W

    pltpu.sync_copy(labels_hbm.at[pl.ds(base, _B_PER_W)], labels_v)
    lane = lax.iota(jnp.int32, _G)
    for j in range(_NCHUNK):
        for k in range(_CHUNK // _G):
            lbl = labels_v[pl.ds(j * _CHUNK + k * _G, _G)]
            pr = lax.shift_left(lax.shift_right_logical(lbl, 11), 10)
            idx_v[j, pl.ds(k * _G, _G)] = pr | (lbl & 1023)
    copies = []
    for j in range(_NCHUNK):
        copies.append(
            pltpu.async_copy(
                pairs_hbm.at[idx_v.at[j]],
                rows_v.at[pl.ds(j * _CHUNK, _CHUNK)],
                sem,
            )
        )
    pltpu.sync_copy(feat_t.at[:, pl.ds(base, _B_PER_W)], featv)
    pltpu.sync_copy(streak_hbm.at[pl.ds(base, _B_PER_W)], streak_v)
    for c in copies:
        c.wait()

    def group(g, _):
        loc = g * _G
        slot = loc + lane
        half = (lax.shift_right_logical(labels_v[pl.ds(loc, _G)], 10) & 1) * EMBED_DIM
        dot = jnp.zeros((_G,), jnp.float32)
        nrm = jnp.zeros((_G,), jnp.float32)
        for d in range(EMBED_DIM):
            e = plsc.load_gather(rows_v, [slot, half + d])
            f = featv[d, pl.ds(loc, _G)]
            dot = dot + e * f
            nrm = nrm + f * f
        ns = jnp.maximum(nrm, 1e-30)
        norm = jnp.maximum(ns * _rsqrt(ns), 1e-6)
        alpha = 1.0 / (1.0 + jnp.exp(-(dot / (SCALE * norm))))
        amp = 1.0 + LAMBDA_MCE * (1.0 - alpha) * streak_v[pl.ds(loc, _G)]
        outv[pl.ds(loc, _G)] = amp
        return ()

    lax.fori_loop(0, _NGROUP, group, (), unroll=False)

    pltpu.sync_copy(outv, out_hbm.at[0, pl.ds(base, _B_PER_W)])
    pltpu.sync_copy(outv, out_hbm.at[1, pl.ds(base, _B_PER_W)])


def _tc_pairs_body(x1_ref, x2_ref, o_ref):
    x = jnp.concatenate([x1_ref[...], x2_ref[...]], axis=0)
    o_ref[...] = jnp.transpose(x)


_LASTB = 2 * _TGRID - 2


def _tc_pairs(table_t):
    return pl.pallas_call(
        _tc_pairs_body,
        grid=(_TGRID,),
        in_specs=[
            pl.BlockSpec((EMBED_DIM, _CB), lambda i: (0, jnp.minimum(2 * i, _LASTB))),
            pl.BlockSpec((EMBED_DIM, _CB), lambda i: (0, jnp.minimum(2 * i + 1, _LASTB))),
        ],
        out_specs=pl.BlockSpec((_CB, 2 * EMBED_DIM), lambda i: (i, 0)),
        out_shape=jax.ShapeDtypeStruct((_PAIR_ROWS, 2 * EMBED_DIM), jnp.float32),
    )(table_t, table_t)


def kernel(gt_labels, pooled_features, streak_ratio, table):
    table_t = jnp.swapaxes(table, 0, 1)
    pairs = _tc_pairs(table_t)
    feat_t = jnp.swapaxes(pooled_features, 0, 1)
    return _sc_fused(pairs, feat_t, gt_labels, streak_ratio)
